# baseline (speedup 1.0000x reference)
import functools

import jax
import jax.numpy as jnp
from jax import lax
from jax.experimental import pallas as pl
from jax.experimental.pallas import tpu as pltpu
from jax.experimental.pallas import tpu_sc as plsc

N_NODES = 10000
D = 256
DH = 128
N_EDGES = 160000
N_SUB = 16
CHUNK = 128
NCH = 81
NCH_STG = 90
E_TILE = N_EDGES // N_SUB
PAD_TILE = NCH * CHUNK - E_TILE
ACC_ROWS = 10008
TRASH = N_NODES
INIT_A = 624

_mesh = plsc.VectorSubcoreMesh(core_axis_name="c", subcore_axis_name="s")


@functools.partial(
    pl.kernel,
    out_type=jax.ShapeDtypeStruct((2, ACC_ROWS, DH), jnp.float32),
    mesh=_mesh,
    scratch_types=[
        pltpu.VMEM((3, 1, 3 * CHUNK), jnp.int32),
        pltpu.VMEM((3, CHUNK), jnp.int32),
        pltpu.VMEM((1, CHUNK), jnp.int32),
        pltpu.VMEM((CHUNK, DH), jnp.float32),
        pltpu.VMEM((CHUNK, DH), jnp.float32),
        pltpu.VMEM((CHUNK, DH), jnp.float32),
        pltpu.VMEM_SHARED((ACC_ROWS, DH), jnp.float32),
        pltpu.SemaphoreType.DMA,
        pltpu.SemaphoreType.DMA,
        pltpu.SemaphoreType.DMA,
        pltpu.SemaphoreType.DMA,
        pltpu.SemaphoreType.DMA,
        pltpu.SemaphoreType.DMA,
    ],
)
def _sc_aggregate(x_hbm, sd_hbm, out_hbm, ring, src_x, dst_x,
                  rows0, rows1, rows2, acc_sh, g0, g1, g2, t0, t1, t2):
    c = lax.axis_index("c")
    s = lax.axis_index("s")
    col = c * DH
    rows = (rows0, rows1, rows2)
    gsem = (g0, g1, g2)
    ssem = (t0, t1, t2)

    @pl.when(s < N_SUB - 1)
    def _():
        pltpu.sync_copy(
            x_hbm.at[pl.ds(s * INIT_A, INIT_A), pl.ds(col, DH)],
            acc_sh.at[pl.ds(s * INIT_A, INIT_A)],
        )

    @pl.when(s == N_SUB - 1)
    def _():
        pltpu.sync_copy(
            x_hbm.at[pl.ds((N_SUB - 1) * INIT_A, 640), pl.ds(col, DH)],
            acc_sh.at[pl.ds((N_SUB - 1) * INIT_A, 640)],
        )
    plsc.subcore_barrier()

    def stage(p, slot):
        pltpu.async_copy(sd_hbm.at[s, pl.ds(3 * CHUNK * p, 3 * CHUNK)],
                         ring.at[slot, 0], ssem[slot])

    def wait_stage(slot):
        pltpu.make_async_copy(sd_hbm.at[s, pl.ds(0, 3 * CHUNK)],
                              ring.at[slot, 0], ssem[slot]).wait()

    def extract_src(slot, b, xslot):
        for l in range(8):
            v = ring[slot, 0, pl.ds(b * CHUNK + l * 16, 16)]
            src_x[xslot, pl.ds(l * 16, 16)] = jnp.bitwise_and(v, 0xFFFF)

    def extract_dst(slot, b):
        for l in range(8):
            v = ring[slot, 0, pl.ds(b * CHUNK + l * 16, 16)]
            dst_x[0, pl.ds(l * 16, 16)] = lax.shift_right_logical(v, 16)

    def fire(buf):
        pltpu.async_copy(x_hbm.at[src_x.at[buf], pl.ds(col, DH)],
                         rows[buf], gsem[buf])

    def wait_g(buf):
        pltpu.make_async_copy(x_hbm.at[src_x.at[buf], pl.ds(col, DH)],
                              rows[buf], gsem[buf]).wait()

    def scatter(buf):
        pltpu.sync_copy(rows[buf], acc_sh.at[dst_x.at[0]], add=True)

    for slot in range(3):
        stage(slot, slot)
    wait_stage(0)
    for b in range(3):
        extract_src(0, b, b)
        fire(b)

    def body(k, carry):
        for jj in range(9):
            p_off, b = jj // 3, jj % 3
            if b == 0:
                wait_stage((p_off + 1) % 3)
            wait_g(b)
            extract_dst(p_off, b)
            scatter(b)
            extract_src((p_off + 1) % 3, b, b)
            fire(b)
            if b == 2:
                stage(3 * k + p_off + 3, p_off)
        return carry

    lax.fori_loop(0, NCH // 9, body, 0)
    for b in range(3):
        wait_g(b)
    wait_stage(1)
    wait_stage(2)
    plsc.subcore_barrier()

    @pl.when(s < N_SUB - 1)
    def _():
        pltpu.sync_copy(
            acc_sh.at[pl.ds(s * INIT_A, INIT_A)],
            out_hbm.at[c, pl.ds(s * INIT_A, INIT_A)],
        )

    @pl.when(s == N_SUB - 1)
    def _():
        pltpu.sync_copy(
            acc_sh.at[pl.ds((N_SUB - 1) * INIT_A, 648)],
            out_hbm.at[c, pl.ds((N_SUB - 1) * INIT_A, 648)],
        )


BLK = 1000


def _mlp_body(hl_ref, hr_ref, w1_ref, b1_ref, w2_ref, b2_ref, o_ref):
    h = jnp.concatenate([hl_ref[0], hr_ref[0]], axis=-1)
    a = jnp.dot(h, w1_ref[...], preferred_element_type=jnp.float32)
    a = jnp.maximum(a + b1_ref[...], 0.0)
    o = jnp.dot(a, w2_ref[...], preferred_element_type=jnp.float32)
    o_ref[...] = jnp.tanh(o + b2_ref[...])


def _mlp(h2, W1, b1, W2, b2):
    return pl.pallas_call(
        _mlp_body,
        grid=(N_NODES // BLK,),
        in_specs=[
            pl.BlockSpec((1, BLK, DH), lambda i: (0, i, 0)),
            pl.BlockSpec((1, BLK, DH), lambda i: (1, i, 0)),
            pl.BlockSpec((D, D), lambda i: (0, 0)),
            pl.BlockSpec((1, D), lambda i: (0, 0)),
            pl.BlockSpec((D, D), lambda i: (0, 0)),
            pl.BlockSpec((1, D), lambda i: (0, 0)),
        ],
        out_specs=pl.BlockSpec((BLK, D), lambda i: (i, 0)),
        out_shape=jax.ShapeDtypeStruct((N_NODES, D), jnp.float32),
    )(h2, h2, W1, b1.reshape(1, D), W2, b2.reshape(1, D))


def kernel(x, edge_index, W1, b1, W2, b2):
    src = edge_index[0].astype(jnp.int32)
    dst = edge_index[1].astype(jnp.int32)
    sd = jnp.bitwise_or(src, dst << 16).reshape(N_SUB, E_TILE)
    i = jnp.arange(PAD_TILE, dtype=jnp.int32)
    pad_sd = jnp.bitwise_or(i, (TRASH + i % (ACC_ROWS - N_NODES)) << 16)
    sd_full = jnp.concatenate([
        sd,
        jnp.broadcast_to(pad_sd, (N_SUB, PAD_TILE)),
        jnp.zeros((N_SUB, (NCH_STG - NCH) * CHUNK), jnp.int32),
    ], axis=1)
    h2 = _sc_aggregate(x, sd_full)
    return _mlp(h2, W1, b1, W2, b2)


# speedup vs baseline: 2.5060x; 2.5060x over previous
import functools

import jax
import jax.numpy as jnp
from jax import lax
from jax.experimental import pallas as pl
from jax.experimental.pallas import tpu as pltpu
from jax.experimental.pallas import tpu_sc as plsc

N_NODES = 10000
D = 256
DH = 128
N_EDGES = 160000
N_SUB = 16
CHUNK = 128
NCH = 80
N_PHASES = 2
E_PAD = N_SUB * NCH * CHUNK
ROWS_PER_SUB = 632
ACC_ROWS = N_SUB * ROWS_PER_SUB
TRASH = N_NODES
X2_ROWS = 2 * N_NODES + (ACC_ROWS - N_NODES)

_mesh = plsc.VectorSubcoreMesh(core_axis_name="c", subcore_axis_name="s")


@functools.partial(
    pl.kernel,
    out_type=jax.ShapeDtypeStruct((2, ACC_ROWS, DH), jnp.float32),
    mesh=_mesh,
    scratch_types=[
        pltpu.VMEM((NCH // N_PHASES, CHUNK), jnp.int32),
        pltpu.VMEM((NCH // N_PHASES, CHUNK), jnp.int32),
        pltpu.VMEM((CHUNK, DH), jnp.float32),
        pltpu.VMEM((CHUNK, DH), jnp.float32),
        pltpu.VMEM_SHARED((ACC_ROWS, DH), jnp.float32),
        pltpu.SemaphoreType.DMA,
        pltpu.SemaphoreType.DMA,
    ],
)
def _sc_aggregate(x_hbm, srcs_hbm, dsts_hbm, out_hbm,
                  src_v, dst_v, rows0, rows1, acc_sh, sem0, sem1):
    c = lax.axis_index("c")
    s = lax.axis_index("s")
    col = c * DH
    @pl.when(s < N_SUB - 1)
    def _():
        pltpu.sync_copy(
            x_hbm.at[pl.ds(s * 624, 624), pl.ds(col, DH)],
            acc_sh.at[pl.ds(s * 624, 624)],
        )

    @pl.when(s == N_SUB - 1)
    def _():
        pltpu.sync_copy(
            x_hbm.at[pl.ds((N_SUB - 1) * 624, 640), pl.ds(col, DH)],
            acc_sh.at[pl.ds((N_SUB - 1) * 624, 640)],
        )
    plsc.subcore_barrier()

    def gather(j, buf, sem):
        return pltpu.async_copy(
            x_hbm.at[src_v.at[j], pl.ds(col, DH)], buf, sem)

    def wait_gather(j, buf, sem):
        pltpu.make_async_copy(
            x_hbm.at[src_v.at[j], pl.ds(col, DH)], buf, sem).wait()

    def scatter_add(j, buf):
        pltpu.sync_copy(buf, acc_sh.at[dst_v.at[j]], add=True)

    P_NCH = NCH // N_PHASES
    for p in range(N_PHASES):
        pltpu.sync_copy(srcs_hbm.at[s, pl.ds(p * P_NCH, P_NCH)], src_v)
        pltpu.sync_copy(dsts_hbm.at[s, pl.ds(p * P_NCH, P_NCH)], dst_v)
        gather(0, rows0, sem0)

        def body(k, carry):
            j = 2 * k
            cp = gather(j + 1, rows1, sem1)
            wait_gather(j, rows0, sem0)
            scatter_add(j, rows0)
            gather(j + 2, rows0, sem0)
            cp.wait()
            scatter_add(j + 1, rows1)
            return carry

        lax.fori_loop(0, P_NCH // 2 - 1, body, 0)
        cp = gather(P_NCH - 1, rows1, sem1)
        wait_gather(P_NCH - 2, rows0, sem0)
        scatter_add(P_NCH - 2, rows0)
        cp.wait()
        scatter_add(P_NCH - 1, rows1)
    plsc.subcore_barrier()
    pltpu.sync_copy(
        acc_sh.at[pl.ds(s * ROWS_PER_SUB, ROWS_PER_SUB)],
        out_hbm.at[c, pl.ds(s * ROWS_PER_SUB, ROWS_PER_SUB)],
    )


BLK = 1000


def _mlp_body(hl_ref, hr_ref, w1_ref, b1_ref, w2_ref, b2_ref, o_ref):
    h = jnp.concatenate([hl_ref[0], hr_ref[0]], axis=-1)
    a = jnp.dot(h, w1_ref[...], preferred_element_type=jnp.float32)
    a = jnp.maximum(a + b1_ref[...], 0.0)
    o = jnp.dot(a, w2_ref[...], preferred_element_type=jnp.float32)
    o_ref[...] = jnp.tanh(o + b2_ref[...])


def _mlp(h2, W1, b1, W2, b2):
    return pl.pallas_call(
        _mlp_body,
        grid=(N_NODES // BLK,),
        in_specs=[
            pl.BlockSpec((1, BLK, DH), lambda i: (0, i, 0)),
            pl.BlockSpec((1, BLK, DH), lambda i: (1, i, 0)),
            pl.BlockSpec((D, D), lambda i: (0, 0)),
            pl.BlockSpec((1, D), lambda i: (0, 0)),
            pl.BlockSpec((D, D), lambda i: (0, 0)),
            pl.BlockSpec((1, D), lambda i: (0, 0)),
        ],
        out_specs=pl.BlockSpec((BLK, D), lambda i: (i, 0)),
        out_shape=jax.ShapeDtypeStruct((N_NODES, D), jnp.float32),
    )(h2, h2, W1, b1.reshape(1, D), W2, b2.reshape(1, D))


def kernel(x, edge_index, W1, b1, W2, b2):
    src = edge_index[0].astype(jnp.int32)
    dst = edge_index[1].astype(jnp.int32)
    pad = E_PAD - N_EDGES
    ar = jnp.arange(pad, dtype=jnp.int32)
    srcs = jnp.concatenate([src, ar % N_NODES]).reshape(N_SUB, NCH, CHUNK)
    dsts = jnp.concatenate(
        [dst, TRASH + ar % (ACC_ROWS - N_NODES)]).reshape(N_SUB, NCH, CHUNK)
    h2 = _sc_aggregate(x, srcs, dsts)
    return _mlp(h2, W1, b1, W2, b2)


# speedup vs baseline: 2.5429x; 1.0147x over previous
import functools

import jax
import jax.numpy as jnp
from jax import lax
from jax.experimental import pallas as pl
from jax.experimental.pallas import tpu as pltpu
from jax.experimental.pallas import tpu_sc as plsc

N_NODES = 10000
D = 256
DH = 128
N_EDGES = 160000
N_SUB = 16
CHUNK = 128
NCH = 80
N_PHASES = 2
E_PAD = N_SUB * NCH * CHUNK
ROWS_PER_SUB = 632
ACC_ROWS = N_SUB * ROWS_PER_SUB
TRASH = N_NODES
X2_ROWS = 2 * N_NODES + (ACC_ROWS - N_NODES)

_mesh = plsc.VectorSubcoreMesh(core_axis_name="c", subcore_axis_name="s")


@functools.partial(
    pl.kernel,
    out_type=jax.ShapeDtypeStruct((2, ACC_ROWS, DH), jnp.float32),
    mesh=_mesh,
    scratch_types=[
        pltpu.VMEM((NCH // N_PHASES, CHUNK), jnp.int32),
        pltpu.VMEM((NCH // N_PHASES, CHUNK), jnp.int32),
        pltpu.VMEM((CHUNK, DH), jnp.float32),
        pltpu.VMEM((CHUNK, DH), jnp.float32),
        pltpu.VMEM_SHARED((ACC_ROWS, DH), jnp.float32),
        pltpu.SemaphoreType.DMA,
        pltpu.SemaphoreType.DMA,
    ],
)
def _sc_aggregate(x_hbm, srcs_hbm, dsts_hbm, out_hbm,
                  src_v, dst_v, rows0, rows1, acc_sh, sem0, sem1):
    c = lax.axis_index("c")
    s = lax.axis_index("s")
    col = c * DH
    @pl.when(s < N_SUB - 1)
    def _():
        pltpu.sync_copy(
            x_hbm.at[pl.ds(s * 624, 624), pl.ds(col, DH)],
            acc_sh.at[pl.ds(s * 624, 624)],
        )

    @pl.when(s == N_SUB - 1)
    def _():
        pltpu.sync_copy(
            x_hbm.at[pl.ds((N_SUB - 1) * 624, 640), pl.ds(col, DH)],
            acc_sh.at[pl.ds((N_SUB - 1) * 624, 640)],
        )
    plsc.subcore_barrier()

    def gather(j, buf, sem):
        return pltpu.async_copy(
            x_hbm.at[src_v.at[j], pl.ds(col, DH)], buf, sem)

    def wait_gather(j, buf, sem):
        pltpu.make_async_copy(
            x_hbm.at[src_v.at[j], pl.ds(col, DH)], buf, sem).wait()

    def scatter_add(j, buf):
        pltpu.sync_copy(buf, acc_sh.at[dst_v.at[j]], add=True)

    P_NCH = NCH // N_PHASES
    for p in range(N_PHASES):
        pltpu.sync_copy(srcs_hbm.at[s, pl.ds(p * P_NCH, P_NCH)], src_v)
        pltpu.sync_copy(dsts_hbm.at[s, pl.ds(p * P_NCH, P_NCH)], dst_v)
        gather(0, rows0, sem0)

        def body(k, carry):
            j = 2 * k
            cp = gather(j + 1, rows1, sem1)
            wait_gather(j, rows0, sem0)
            scatter_add(j, rows0)
            gather(j + 2, rows0, sem0)
            cp.wait()
            scatter_add(j + 1, rows1)
            return carry

        lax.fori_loop(0, P_NCH // 2 - 1, body, 0)
        cp = gather(P_NCH - 1, rows1, sem1)
        wait_gather(P_NCH - 2, rows0, sem0)
        scatter_add(P_NCH - 2, rows0)
        cp.wait()
        scatter_add(P_NCH - 1, rows1)
    plsc.subcore_barrier()
    pltpu.sync_copy(
        acc_sh.at[pl.ds(s * ROWS_PER_SUB, ROWS_PER_SUB)],
        out_hbm.at[c, pl.ds(s * ROWS_PER_SUB, ROWS_PER_SUB)],
    )


BLK = 2000


def _mlp_body(hl_ref, hr_ref, w1_ref, b1_ref, w2_ref, b2_ref, o_ref):
    h = jnp.concatenate([hl_ref[0], hr_ref[0]], axis=-1)
    a = jnp.dot(h, w1_ref[...], preferred_element_type=jnp.float32)
    a = jnp.maximum(a + b1_ref[...], 0.0)
    o = jnp.dot(a, w2_ref[...], preferred_element_type=jnp.float32)
    o_ref[...] = jnp.tanh(o + b2_ref[...])


def _mlp(h2, W1, b1, W2, b2):
    return pl.pallas_call(
        _mlp_body,
        grid=(N_NODES // BLK,),
        in_specs=[
            pl.BlockSpec((1, BLK, DH), lambda i: (0, i, 0)),
            pl.BlockSpec((1, BLK, DH), lambda i: (1, i, 0)),
            pl.BlockSpec((D, D), lambda i: (0, 0)),
            pl.BlockSpec((1, D), lambda i: (0, 0)),
            pl.BlockSpec((D, D), lambda i: (0, 0)),
            pl.BlockSpec((1, D), lambda i: (0, 0)),
        ],
        out_specs=pl.BlockSpec((BLK, D), lambda i: (i, 0)),
        out_shape=jax.ShapeDtypeStruct((N_NODES, D), jnp.float32),
    )(h2, h2, W1, b1.reshape(1, D), W2, b2.reshape(1, D))


def kernel(x, edge_index, W1, b1, W2, b2):
    src = edge_index[0].astype(jnp.int32)
    dst = edge_index[1].astype(jnp.int32)
    pad = E_PAD - N_EDGES
    ar = jnp.arange(pad, dtype=jnp.int32)
    srcs = jnp.concatenate([src, ar % N_NODES]).reshape(N_SUB, NCH, CHUNK)
    dsts = jnp.concatenate(
        [dst, TRASH + ar % (ACC_ROWS - N_NODES)]).reshape(N_SUB, NCH, CHUNK)
    h2 = _sc_aggregate(x, srcs, dsts)
    return _mlp(h2, W1, b1, W2, b2)
